# Initial kernel scaffold; baseline (speedup 1.0000x reference)
import jax
import jax.numpy as jnp
from jax.experimental import pallas as pl


def kernel(x, embeddings, count):
    raise NotImplementedError("write your pallas kernel here")


# speedup vs baseline: 2.3755x; 2.3755x over previous
import functools

import jax
import jax.numpy as jnp
from jax.experimental import pallas as pl
from jax.experimental.pallas import tpu as pltpu

_BS, _TPD, _D = 16384, 4, 32
_G, _K = 4, 512
_ROWS = 1024
_NBLK = _BS // _ROWS


def _vq_body(x_ref, emb_ref, cnt_in_ref, xq_ref, idx_ref, cnt_ref):
    i = pl.program_id(0)
    xb = x_ref[...]
    idx_cols = []
    xq_cols = []
    cnt_rows = []
    for g in range(_G):
        xg = xb[:, g * _D:(g + 1) * _D]
        eg = emb_ref[g]
        cross = jax.lax.dot_general(
            xg, eg, (((1,), (1,)), ((), ())),
            preferred_element_type=jnp.float32)
        x_sq = jnp.sum(xg * xg, axis=1, keepdims=True)
        e_sq = jnp.sum(eg * eg, axis=1)
        d2 = jnp.maximum(x_sq - 2.0 * cross + e_sq[None, :], 0.0)
        idxg = jnp.argmin(d2, axis=1).astype(jnp.int32)
        onehot = (idxg[:, None] == jax.lax.broadcasted_iota(
            jnp.int32, (_ROWS, _K), 1)).astype(jnp.float32)
        xq_cols.append(jax.lax.dot_general(
            onehot, eg, (((1,), (0,)), ((), ())),
            preferred_element_type=jnp.float32))
        idx_cols.append(idxg.reshape(_ROWS, 1))
        cnt_rows.append(jnp.sum(onehot, axis=0).reshape(1, _K))
    xq_ref[...] = jnp.concatenate(xq_cols, axis=1)
    idx_ref[...] = jnp.concatenate(idx_cols, axis=1)
    cnt_blk = jnp.concatenate(cnt_rows, axis=0)

    @pl.when(i == 0)
    def _():
        cnt_ref[...] = cnt_in_ref[...]

    cnt_ref[...] += cnt_blk


@jax.jit
def kernel(x, embeddings, count):
    x2 = x.reshape(_BS, _G * _D)
    grid = (_NBLK,)
    xq, idx, cnt = pl.pallas_call(
        _vq_body,
        grid=grid,
        in_specs=[
            pl.BlockSpec((_ROWS, _G * _D), lambda i: (i, 0)),
            pl.BlockSpec((_G, _K, _D), lambda i: (0, 0, 0)),
            pl.BlockSpec((_G, _K), lambda i: (0, 0)),
        ],
        out_specs=[
            pl.BlockSpec((_ROWS, _G * _D), lambda i: (i, 0)),
            pl.BlockSpec((_ROWS, _G), lambda i: (i, 0)),
            pl.BlockSpec((_G, _K), lambda i: (0, 0)),
        ],
        out_shape=[
            jax.ShapeDtypeStruct((_BS, _G * _D), jnp.float32),
            jax.ShapeDtypeStruct((_BS, _G), jnp.int32),
            jax.ShapeDtypeStruct((_G, _K), jnp.float32),
        ],
    )(x2, embeddings, count)
    return xq.reshape(_BS, _TPD, _D), idx, cnt
